# baseline (speedup 1.0000x reference)
import functools

import jax
import jax.numpy as jnp
from jax import lax
from jax.experimental import pallas as pl
from jax.experimental.pallas import tpu as pltpu
from jax.experimental.pallas import tpu_sc as plsc

N = 10000
C = 128
H = 32
G = 8
CpG = 16

NC = 2
NS = 16
NW = NC * NS

QC = 8
QPW = 320
NP = NW * QPW
NCHUNK = QPW // QC
NTAB = 10240


def _mm_body(x_ref, wt_ref, b_ref, o_ref):
    o_ref[...] = (
        jnp.dot(x_ref[...], wt_ref[...], preferred_element_type=jnp.float32)
        + b_ref[...]
    )


def _attention_mm(x, wt, b2):
    m = x.shape[0]
    bm = 512
    return pl.pallas_call(
        _mm_body,
        grid=(m // bm,),
        in_specs=[
            pl.BlockSpec((bm, C), lambda i: (i, 0)),
            pl.BlockSpec((C, H * CpG), lambda i: (0, 0)),
            pl.BlockSpec((1, H * CpG), lambda i: (0, 0)),
        ],
        out_specs=pl.BlockSpec((bm, H * CpG), lambda i: (i, 0)),
        out_shape=jax.ShapeDtypeStruct((m, H * CpG), jnp.float32),
    )(x, wt, b2)


def _sc_body(sf_hbm, aw_hbm, idx_hbm, out_hbm,
             sf_spm, idx_v, rows_v, aw_v, out_v,
             isem0, isem1, gsem0, gsem1, asem0, asem1, osem0, osem1):
    sid = lax.axis_index("s")
    wid = sid * NC + lax.axis_index("c")

    rows_per = NTAB // NS
    pltpu.sync_copy(
        sf_hbm.at[pl.ds(sid * rows_per, rows_per)],
        sf_spm.at[pl.ds(sid * rows_per, rows_per)],
    )
    plsc.subcore_barrier()
    qbase = wid * QPW
    isems = (isem0, isem1)
    gsems = (gsem0, gsem1)
    asems = (asem0, asem1)
    osems = (osem0, osem1)

    def edge_off(i):
        return pl.multiple_of((qbase + i * QC) * H, QC * H)

    def idx_pair(i, s):
        return idx_hbm.at[pl.ds(edge_off(i), QC * H)], idx_v.at[s]

    def aw_pair(i, s):
        return aw_hbm.at[pl.ds(i * QC + qbase, QC)], aw_v.at[s]

    def gather_pair(s, k):
        return (
            sf_spm.at[idx_v.at[s, pl.ds(k * 128, 128)]],
            rows_v.at[s, k],
        )

    def out_pair(i, s):
        return out_v.at[s], out_hbm.at[pl.ds(qbase + i * QC, QC)]

    def compute(s):
        def hbody(h, acc):
            new = []
            for q in range(QC):
                r = (q % 4) * H + h
                w = aw_v[s, q, pl.ds(h * CpG, CpG)]
                for j in range(G // 2):
                    packed = rows_v[s, q // 4, r, pl.ds(j * CpG, CpG)]
                    a = lax.bitcast_convert_type(packed << 16, jnp.float32)
                    b2 = lax.bitcast_convert_type(packed, jnp.float32)
                    new.append(acc[q * G + 2 * j] + a * w)
                    new.append(acc[q * G + 2 * j + 1] + b2 * w)
            return tuple(new)

        acc = lax.fori_loop(
            0, H, hbody, (jnp.zeros((CpG,), jnp.float32),) * (QC * G)
        )
        for q in range(QC):
            for j in range(G):
                out_v[s, q, pl.ds(j * CpG, CpG)] = acc[q * G + j]

    def chunk(i, b, pf_next, pf_idx, w_out):
        pltpu.make_async_copy(*gather_pair(b, 0), gsems[b]).wait()
        pltpu.make_async_copy(*gather_pair(b, 1), gsems[b]).wait()
        pltpu.make_async_copy(*aw_pair(i, b), asems[b]).wait()
        if pf_next:
            pltpu.make_async_copy(*idx_pair(i + 1, 1 - b), isems[1 - b]).wait()
            pltpu.async_copy(*gather_pair(1 - b, 0), gsems[1 - b])
            pltpu.async_copy(*gather_pair(1 - b, 1), gsems[1 - b])
            pltpu.async_copy(*aw_pair(i + 1, 1 - b), asems[1 - b])
        if pf_idx:
            pltpu.async_copy(*idx_pair(i + 2, b), isems[b])
        if w_out:
            pltpu.make_async_copy(*out_pair(i, b), osems[b]).wait()
        compute(b)
        pltpu.async_copy(*out_pair(i, b), osems[b])

    pltpu.async_copy(*idx_pair(0, 0), isems[0])
    pltpu.async_copy(*idx_pair(1, 1), isems[1])
    pltpu.make_async_copy(*idx_pair(0, 0), isems[0]).wait()
    pltpu.async_copy(*gather_pair(0, 0), gsems[0])
    pltpu.async_copy(*gather_pair(0, 1), gsems[0])
    pltpu.async_copy(*aw_pair(0, 0), asems[0])

    chunk(0, 0, True, True, False)
    chunk(1, 1, True, True, False)

    def steady(g, carry):
        i = 2 * g
        chunk(i, 0, True, True, True)
        chunk(i + 1, 1, True, True, True)
        return carry

    lax.fori_loop(1, NCHUNK // 2 - 1, steady, 0)

    chunk(NCHUNK - 2, 0, True, False, True)
    chunk(NCHUNK - 1, 1, False, False, True)
    pltpu.make_async_copy(*out_pair(NCHUNK - 2, 0), osems[0]).wait()
    pltpu.make_async_copy(*out_pair(NCHUNK - 1, 1), osems[1]).wait()


_sc_call = pl.kernel(
    _sc_body,
    out_type=jax.ShapeDtypeStruct((NP, C), jnp.float32),
    mesh=plsc.VectorSubcoreMesh(
        core_axis_name="c", subcore_axis_name="s", num_cores=NC, num_subcores=NS
    ),
    scratch_types=[
        pltpu.VMEM_SHARED((NTAB, C // 2), jnp.int32),
        pltpu.VMEM((2, QC * H), jnp.int32),
        pltpu.VMEM((2, 2, QC * H // 2, C // 2), jnp.int32),
        pltpu.VMEM((2, QC, H * CpG), jnp.float32),
        pltpu.VMEM((2, QC, C), jnp.float32),
        pltpu.SemaphoreType.DMA,
        pltpu.SemaphoreType.DMA,
        pltpu.SemaphoreType.DMA,
        pltpu.SemaphoreType.DMA,
        pltpu.SemaphoreType.DMA,
        pltpu.SemaphoreType.DMA,
        pltpu.SemaphoreType.DMA,
        pltpu.SemaphoreType.DMA,
    ],
    compiler_params=pltpu.CompilerParams(use_tc_tiling_on_sc=False),
)


def kernel(q_pts, s_pts, s_feats, neighb_inds, W, b):
    sf_pair = (
        s_feats.reshape(N, CpG, G // 2, 2)
        .transpose(0, 2, 1, 3)
        .astype(jnp.bfloat16)
    )
    sf_t = jnp.pad(
        jax.lax.bitcast_convert_type(sf_pair, jnp.int32).reshape(N, C // 2),
        ((0, NTAB - N), (0, 0)),
    )
    x_pad = jnp.pad(s_feats, ((0, NP - N), (0, 0)))
    aw = _attention_mm(x_pad, W.T, b.reshape(1, H * CpG))
    idx_flat = jnp.pad(neighb_inds, ((0, NP - N), (0, 0))).reshape(NP * H)
    out_t = _sc_call(sf_t, aw, idx_flat)
    return out_t[:N].reshape(N, G, CpG).transpose(0, 2, 1).reshape(N, C)


# speedup vs baseline: 2.3205x; 2.3205x over previous
import functools

import jax
import jax.numpy as jnp
from jax import lax
from jax.experimental import pallas as pl
from jax.experimental.pallas import tpu as pltpu
from jax.experimental.pallas import tpu_sc as plsc

N = 10000
C = 128
H = 32
G = 8
CpG = 16

NC = 2
NS = 16
NW = NC * NS

QC = 4
QPW = 320
NP = NW * QPW
NCHUNK = QPW // QC
NTAB = 10240


def _mm_body(x_ref, wt_ref, b_ref, o_ref):
    o_ref[...] = (
        jnp.dot(x_ref[...], wt_ref[...], preferred_element_type=jnp.float32)
        + b_ref[...]
    )


def _attention_mm(x, wt, b2):
    m = x.shape[0]
    bm = 1000
    return pl.pallas_call(
        _mm_body,
        grid=(m // bm,),
        in_specs=[
            pl.BlockSpec((bm, C), lambda i: (i, 0)),
            pl.BlockSpec((C, H * CpG), lambda i: (0, 0)),
            pl.BlockSpec((1, H * CpG), lambda i: (0, 0)),
        ],
        out_specs=pl.BlockSpec((bm, H * CpG), lambda i: (i, 0)),
        out_shape=jax.ShapeDtypeStruct((m, H * CpG), jnp.float32),
    )(x, wt, b2)


def _sc_body(sf_hbm, aw_hbm, idx_hbm, out_hbm,
             sf_spm, idx_v, rows_v, aw_v, out_v,
             isem0, isem1, gsem0, gsem1, asem0, asem1, osem0, osem1):
    sid = lax.axis_index("s")
    wid = sid * NC + lax.axis_index("c")

    rows_per = NTAB // NS
    pltpu.sync_copy(
        sf_hbm.at[pl.ds(sid * rows_per, rows_per)],
        sf_spm.at[pl.ds(sid * rows_per, rows_per)],
    )
    plsc.subcore_barrier()
    qbase = wid * QPW
    isems = (isem0, isem1)
    gsems = (gsem0, gsem1)
    asems = (asem0, asem1)
    osems = (osem0, osem1)

    def q0_of(i):
        return jnp.minimum(qbase + i * QC, N - QC)

    def edge_off(i):
        return pl.multiple_of(q0_of(i) * H, QC * H)

    def idx_pair(i, s):
        return idx_hbm.at[pl.ds(edge_off(i), QC * H)], idx_v.at[s]

    def aw_pair(i, s):
        return aw_hbm.at[pl.ds(q0_of(i), QC)], aw_v.at[s]

    def gather_pair(s):
        return sf_spm.at[idx_v.at[s]], rows_v.at[s]

    def out_pair(i, s):
        return out_v.at[s], out_hbm.at[pl.ds(q0_of(i), QC)]

    def compute(s):
        def hbody(h, acc):
            new = []
            for q in range(QC):
                r = q * H + h
                w = aw_v[s, q, pl.ds(h * CpG, CpG)]
                for j in range(G // 2):
                    packed = rows_v[s, r, pl.ds(j * CpG, CpG)]
                    a = lax.bitcast_convert_type(packed << 16, jnp.float32)
                    b2 = lax.bitcast_convert_type(packed, jnp.float32)
                    new.append(acc[q * G + 2 * j] + a * w)
                    new.append(acc[q * G + 2 * j + 1] + b2 * w)
            return tuple(new)

        acc = lax.fori_loop(
            0, H, hbody, (jnp.zeros((CpG,), jnp.float32),) * (QC * G)
        )
        for q in range(QC):
            for j in range(G):
                out_v[s, q, pl.ds(j * CpG, CpG)] = acc[q * G + j]

    def chunk(i, b, pf_next, pf_idx, w_out):
        pltpu.make_async_copy(*gather_pair(b), gsems[b]).wait()
        pltpu.make_async_copy(*aw_pair(i, b), asems[b]).wait()
        if pf_next:
            pltpu.make_async_copy(*idx_pair(i + 1, 1 - b), isems[1 - b]).wait()
            pltpu.async_copy(*gather_pair(1 - b), gsems[1 - b])
            pltpu.async_copy(*aw_pair(i + 1, 1 - b), asems[1 - b])
        if pf_idx:
            pltpu.async_copy(*idx_pair(i + 2, b), isems[b])
        if w_out:
            pltpu.make_async_copy(*out_pair(i, b), osems[b]).wait()
        compute(b)
        pltpu.async_copy(*out_pair(i, b), osems[b])

    pltpu.async_copy(*idx_pair(0, 0), isems[0])
    pltpu.async_copy(*idx_pair(1, 1), isems[1])
    pltpu.make_async_copy(*idx_pair(0, 0), isems[0]).wait()
    pltpu.async_copy(*gather_pair(0), gsems[0])
    pltpu.async_copy(*aw_pair(0, 0), asems[0])

    chunk(0, 0, True, True, False)
    chunk(1, 1, True, True, False)

    def steady(g, carry):
        i = 2 * g
        chunk(i, 0, True, True, True)
        chunk(i + 1, 1, True, True, True)
        return carry

    lax.fori_loop(1, NCHUNK // 2 - 1, steady, 0)

    chunk(NCHUNK - 2, 0, True, False, True)
    chunk(NCHUNK - 1, 1, False, False, True)
    pltpu.make_async_copy(*out_pair(NCHUNK - 2, 0), osems[0]).wait()
    pltpu.make_async_copy(*out_pair(NCHUNK - 1, 1), osems[1]).wait()


_sc_call = pl.kernel(
    _sc_body,
    out_type=jax.ShapeDtypeStruct((N, C), jnp.float32),
    mesh=plsc.VectorSubcoreMesh(
        core_axis_name="c", subcore_axis_name="s", num_cores=NC, num_subcores=NS
    ),
    scratch_types=[
        pltpu.VMEM_SHARED((NTAB, C // 2), jnp.int32),
        pltpu.VMEM((2, QC * H), jnp.int32),
        pltpu.VMEM((2, QC * H, C // 2), jnp.int32),
        pltpu.VMEM((2, QC, H * CpG), jnp.float32),
        pltpu.VMEM((2, QC, C), jnp.float32),
        pltpu.SemaphoreType.DMA,
        pltpu.SemaphoreType.DMA,
        pltpu.SemaphoreType.DMA,
        pltpu.SemaphoreType.DMA,
        pltpu.SemaphoreType.DMA,
        pltpu.SemaphoreType.DMA,
        pltpu.SemaphoreType.DMA,
        pltpu.SemaphoreType.DMA,
    ],
    compiler_params=pltpu.CompilerParams(use_tc_tiling_on_sc=False),
)


def kernel(q_pts, s_pts, s_feats, neighb_inds, W, b):
    sf_pair = (
        s_feats.reshape(N, CpG, G // 2, 2)
        .transpose(0, 2, 1, 3)
        .astype(jnp.bfloat16)
    )
    sf_t = jnp.pad(
        jax.lax.bitcast_convert_type(sf_pair, jnp.int32).reshape(N, C // 2),
        ((0, NTAB - N), (0, 0)),
    )
    aw = _attention_mm(s_feats, W.T, b.reshape(1, H * CpG))
    idx_flat = neighb_inds.reshape(N * H)
    out_t = _sc_call(sf_t, aw, idx_flat)
    return out_t.reshape(N, G, CpG).transpose(0, 2, 1).reshape(N, C)


# speedup vs baseline: 2.3418x; 1.0092x over previous
import jax
import jax.numpy as jnp
from jax import lax
from jax.experimental import pallas as pl
from jax.experimental.pallas import tpu as pltpu
from jax.experimental.pallas import tpu_sc as plsc

N = 10000
C = 128
H = 32
G = 8
CpG = 16

NC = 2
NS = 16
NW = NC * NS

QC = 4
QPW = 320
NCHUNK = QPW // QC
NTAB = 10240


def _mm_body(x_ref, wt_ref, b_ref, o_ref):
    o_ref[...] = (
        jnp.dot(x_ref[...], wt_ref[...], preferred_element_type=jnp.float32)
        + b_ref[...]
    )


def _attention_mm(x, wt, b2):
    m = x.shape[0]
    bm = 1000
    return pl.pallas_call(
        _mm_body,
        grid=(m // bm,),
        in_specs=[
            pl.BlockSpec((bm, C), lambda i: (i, 0)),
            pl.BlockSpec((C, H * CpG), lambda i: (0, 0)),
            pl.BlockSpec((1, H * CpG), lambda i: (0, 0)),
        ],
        out_specs=pl.BlockSpec((bm, H * CpG), lambda i: (i, 0)),
        out_shape=jax.ShapeDtypeStruct((m, H * CpG), jnp.float32),
    )(x, wt, b2)


def _sc_body(sf_hbm, aw_hbm, idx_hbm, out_hbm,
             sf_spm, idx_v, rows_v, aw_v, out_v,
             isem0, isem1, gsem0, gsem1, asem0, asem1, osem0, osem1):
    sid = lax.axis_index("s")
    wid = sid * NC + lax.axis_index("c")

    rows_per = NTAB // NS
    pltpu.sync_copy(
        sf_hbm.at[pl.ds(sid * rows_per, rows_per)],
        sf_spm.at[pl.ds(sid * rows_per, rows_per)],
    )
    plsc.subcore_barrier()
    qbase = wid * QPW
    isems = (isem0, isem1)
    gsems = (gsem0, gsem1)
    asems = (asem0, asem1)
    osems = (osem0, osem1)

    def q0_of(i):
        return jnp.minimum(qbase + i * QC, N - QC)

    def edge_off(i):
        return pl.multiple_of(q0_of(i) * H, QC * H)

    def idx_pair(i, s):
        return idx_hbm.at[pl.ds(edge_off(i), QC * H)], idx_v.at[s]

    def aw_pair(i, s):
        return aw_hbm.at[pl.ds(q0_of(i), QC)], aw_v.at[s]

    def gather_pair(s):
        return sf_spm.at[idx_v.at[s]], rows_v.at[s]

    def out_pair(i, s):
        return out_v.at[s], out_hbm.at[pl.ds(q0_of(i), QC)]

    def compute(s):
        def hbody(h, acc):
            new = []
            for q in range(QC):
                r = q * H + h
                w = aw_v[s, q, pl.ds(h * CpG, CpG)]
                for j in range(G // 2):
                    packed = rows_v[s, r, pl.ds(j * CpG, CpG)]
                    a = lax.bitcast_convert_type(packed << 16, jnp.float32)
                    b2 = lax.bitcast_convert_type(packed, jnp.float32)
                    new.append(acc[q * G + 2 * j] + a * w)
                    new.append(acc[q * G + 2 * j + 1] + b2 * w)
            return tuple(new)

        acc = lax.fori_loop(
            0, H, hbody, (jnp.zeros((CpG,), jnp.float32),) * (QC * G)
        )
        for q in range(QC):
            for j in range(G):
                out_v[s, q, pl.ds(j * CpG, CpG)] = acc[q * G + j]

    def chunk(i, b, pf_next, pf_idx, w_out):
        pltpu.make_async_copy(*gather_pair(b), gsems[b]).wait()
        pltpu.make_async_copy(*aw_pair(i, b), asems[b]).wait()
        if pf_next:
            pltpu.make_async_copy(*idx_pair(i + 1, 1 - b), isems[1 - b]).wait()
            pltpu.async_copy(*gather_pair(1 - b), gsems[1 - b])
            pltpu.async_copy(*aw_pair(i + 1, 1 - b), asems[1 - b])
        if pf_idx:
            pltpu.async_copy(*idx_pair(i + 2, b), isems[b])
        if w_out:
            pltpu.make_async_copy(*out_pair(i, b), osems[b]).wait()
        compute(b)
        pltpu.async_copy(*out_pair(i, b), osems[b])

    pltpu.async_copy(*idx_pair(0, 0), isems[0])
    pltpu.async_copy(*idx_pair(1, 1), isems[1])
    pltpu.make_async_copy(*idx_pair(0, 0), isems[0]).wait()
    pltpu.async_copy(*gather_pair(0), gsems[0])
    pltpu.async_copy(*aw_pair(0, 0), asems[0])

    chunk(0, 0, True, True, False)
    chunk(1, 1, True, True, False)

    def steady(g, carry):
        i = 2 * g
        chunk(i, 0, True, True, True)
        chunk(i + 1, 1, True, True, True)
        return carry

    lax.fori_loop(1, NCHUNK // 2 - 1, steady, 0)

    chunk(NCHUNK - 2, 0, True, False, True)
    chunk(NCHUNK - 1, 1, False, False, True)
    pltpu.make_async_copy(*out_pair(NCHUNK - 2, 0), osems[0]).wait()
    pltpu.make_async_copy(*out_pair(NCHUNK - 1, 1), osems[1]).wait()


_sc_call = pl.kernel(
    _sc_body,
    out_type=jax.ShapeDtypeStruct((N, C), jnp.float32),
    mesh=plsc.VectorSubcoreMesh(
        core_axis_name="c", subcore_axis_name="s", num_cores=NC, num_subcores=NS
    ),
    scratch_types=[
        pltpu.VMEM_SHARED((NTAB, C // 2), jnp.int32),
        pltpu.VMEM((2, QC * H), jnp.int32),
        pltpu.VMEM((2, QC * H, C // 2), jnp.int32),
        pltpu.VMEM((2, QC, H * CpG), jnp.float32),
        pltpu.VMEM((2, QC, C), jnp.float32),
        pltpu.SemaphoreType.DMA,
        pltpu.SemaphoreType.DMA,
        pltpu.SemaphoreType.DMA,
        pltpu.SemaphoreType.DMA,
        pltpu.SemaphoreType.DMA,
        pltpu.SemaphoreType.DMA,
        pltpu.SemaphoreType.DMA,
        pltpu.SemaphoreType.DMA,
    ],
    compiler_params=pltpu.CompilerParams(use_tc_tiling_on_sc=False),
)


def kernel(q_pts, s_pts, s_feats, neighb_inds, W, b):
    sf_pair = (
        s_feats.reshape(N, CpG, G // 2, 2)
        .transpose(0, 2, 1, 3)
        .astype(jnp.bfloat16)
    )
    sf_t = jnp.pad(
        jax.lax.bitcast_convert_type(sf_pair, jnp.int32).reshape(N, C // 2),
        ((0, NTAB - N), (0, 0)),
    )
    aw = _attention_mm(s_feats, W.T, b.reshape(1, H * CpG))
    idx_flat = neighb_inds.reshape(N * H)
    out_t = _sc_call(sf_t, aw, idx_flat)
    return out_t.reshape(N, G, CpG).transpose(0, 2, 1).reshape(N, C)
